# baseline (speedup 1.0000x reference)
import numpy as np
import jax
import jax.numpy as jnp
from jax.experimental import pallas as pl
from jax.experimental.pallas import tpu as pltpu

_FEATURE_STRIDES = [8]
_FEATURE_SHAPES = [(64, 64)]
_ANCHOR_SCALES = np.array([8.0, 16.0, 32.0], dtype=np.float32)
_ANCHOR_RATIOS = np.array([0.5, 1.0, 2.0], dtype=np.float32)
_PRE_NMS_TOP_N = 6000
_POST_NMS_TOP_N = 300
_NMS_THRESH = 0.7
_MIN_SIZE = 16.0
_LOG_MAX = float(np.log(1000.0 / 16.0))

_N = 36864
_R = 288
_C = 128


def _make_base_anchors(base_size, ratios, scales):
    w = h = float(base_size)
    x_ctr = y_ctr = (base_size - 1) * 0.5
    size = w * h
    size_ratios = size / ratios
    ws = np.round(np.sqrt(size_ratios))
    hs = np.round(ws * ratios)
    anchors = []
    for wi, hi in zip(ws, hs):
        for s in scales:
            wss = wi * s
            hss = hi * s
            anchors.append([x_ctr - 0.5 * (wss - 1), y_ctr - 0.5 * (hss - 1),
                            x_ctr + 0.5 * (wss - 1), y_ctr + 0.5 * (hss - 1)])
    return np.array(anchors, dtype=np.float32)


def _make_all_anchors():
    out = []
    for (fh, fw), stride in zip(_FEATURE_SHAPES, _FEATURE_STRIDES):
        base = _make_base_anchors(stride, _ANCHOR_RATIOS, _ANCHOR_SCALES)
        sx = np.arange(fw) * stride
        sy = np.arange(fh) * stride
        gx, gy = np.meshgrid(sx, sy)
        shifts = np.stack([gx.ravel(), gy.ravel(), gx.ravel(), gy.ravel()],
                          axis=1).astype(np.float32)
        out.append((shifts[:, None, :] + base[None, :, :]).reshape(-1, 4))
    return np.concatenate(out, axis=0)


_A = _make_all_anchors()
_WA = (_A[:, 2] - _A[:, 0] + np.float32(1.0)).reshape(_R, _C)
_HA = (_A[:, 3] - _A[:, 1] + np.float32(1.0)).reshape(_R, _C)
_CXA = (_A[:, 0] + np.float32(0.5) * _WA.reshape(-1)).reshape(_R, _C)
_CYA = (_A[:, 1] + np.float32(0.5) * _HA.reshape(-1)).reshape(_R, _C)


def _rpn_body(hw_ref, s_ref, dx_ref, dy_ref, dw_ref, dh_ref,
              wa_ref, ha_ref, cxa_ref, cya_ref, out_ref, ms0, ms1):
    h1 = hw_ref[0, 0]
    w1 = hw_ref[0, 1]
    wa = wa_ref[...]
    ha = ha_ref[...]

    lin = (jax.lax.broadcasted_iota(jnp.int32, (_R, _C), 0) * _C
           + jax.lax.broadcasted_iota(jnp.int32, (_R, _C), 1))

    def decode(b):
        s = s_ref[b]
        dx = dx_ref[b]
        dy = dy_ref[b]
        dw = jnp.minimum(dw_ref[b], _LOG_MAX)
        dh = jnp.minimum(dh_ref[b], _LOG_MAX)
        pcx = dx * wa + cxa_ref[...]
        pcy = dy * ha + cya_ref[...]
        pw = jnp.exp(dw) * wa
        ph = jnp.exp(dh) * ha
        x1 = jnp.clip(pcx - 0.5 * pw, 0.0, w1)
        y1 = jnp.clip(pcy - 0.5 * ph, 0.0, h1)
        x2 = jnp.clip(pcx + 0.5 * pw, 0.0, w1)
        y2 = jnp.clip(pcy + 0.5 * ph, 0.0, h1)
        ws = x2 - x1 + 1.0
        hs = y2 - y1 + 1.0
        s = jnp.where((ws >= _MIN_SIZE) & (hs >= _MIN_SIZE), s, -1.0)
        areas = ws * hs
        ki = jax.lax.bitcast_convert_type(s, jnp.int32)
        key = jnp.where(ki < 0, ki ^ jnp.int32(0x7FFFFFFF), ki)
        return s, x1, y1, x2, y2, areas, key

    sA, x1A, y1A, x2A, y2A, arA, keyA = decode(0)
    sB, x1B, y1B, x2B, y2B, arB, keyB = decode(1)

    def bs1(_, st):
        loA, hiA, loB, hiB = st
        midA = (loA & hiA) + ((loA ^ hiA) >> 1)
        midB = (loB & hiB) + ((loB ^ hiB) >> 1)
        cntA = jnp.sum(jnp.where(keyA > midA, 1.0, 0.0))
        cntB = jnp.sum(jnp.where(keyB > midB, 1.0, 0.0))
        pA = cntA < _PRE_NMS_TOP_N
        pB = cntB < _PRE_NMS_TOP_N
        return (jnp.where(pA, loA, midA), jnp.where(pA, midA, hiA),
                jnp.where(pB, loB, midB), jnp.where(pB, midB, hiB))

    ilo = jnp.int32(-2**31)
    ihi = jnp.int32(2**31 - 1)
    _, tauA, _, tauB = jax.lax.fori_loop(0, 32, bs1, (ilo, ihi, ilo, ihi))
    remA = jnp.float32(_PRE_NMS_TOP_N) - jnp.sum(jnp.where(keyA > tauA, 1.0, 0.0))
    remB = jnp.float32(_PRE_NMS_TOP_N) - jnp.sum(jnp.where(keyB > tauB, 1.0, 0.0))
    eqA = keyA == tauA
    eqB = keyB == tauB

    def bs2(_, st):
        loA, hiA, loB, hiB = st
        midA = (loA & hiA) + ((loA ^ hiA) >> 1)
        midB = (loB & hiB) + ((loB ^ hiB) >> 1)
        cntA = jnp.sum(jnp.where(eqA & (lin <= midA), 1.0, 0.0))
        cntB = jnp.sum(jnp.where(eqB & (lin <= midB), 1.0, 0.0))
        pA = cntA >= remA
        pB = cntB >= remB
        return (jnp.where(pA, loA, midA), jnp.where(pA, midA, hiA),
                jnp.where(pB, loB, midB), jnp.where(pB, midB, hiB))

    m1 = jnp.int32(-1)
    nn = jnp.int32(_N - 1)
    _, isA, _, isB = jax.lax.fori_loop(0, 17, bs2, (m1, nn, m1, nn))

    neg_inf = jnp.float32(-jnp.inf)
    ms0[...] = jnp.where((keyA > tauA) | (eqA & (lin <= isA)), sA, neg_inf)
    ms1[...] = jnp.where((keyB > tauB) | (eqB & (lin <= isB)), sB, neg_inf)

    lane512 = jax.lax.broadcasted_iota(jnp.int32, (1, 512), 1)

    def round2(ms_ref, x1, y1, x2, y2, areas, st):
        ax1, ay1, ax2, ay2, mx8, outcol, alive = st
        m1 = jnp.max(mx8, keepdims=True)
        has1 = m1 > neg_inf
        masked = ms_ref[...]
        hot1 = (masked == m1) & (masked > neg_inf)
        cnt1 = jnp.sum(jnp.where(hot1, 1.0, 0.0))
        wms = jnp.where(hot1, neg_inf, masked)
        m2 = jnp.max(wms.reshape(36, 8, _C), axis=0)
        m2 = jnp.max(m2, keepdims=True)
        has2 = m2 > neg_inf
        hot2 = (wms == m2) & (wms > neg_inf)
        cnt2 = jnp.sum(jnp.where(hot2, 1.0, 0.0))
        tie = (cnt1 > 1.5) | (cnt2 > 1.5)

        def common(_):
            return (jnp.sum(jnp.where(hot1, x1, 0.0), keepdims=True),
                    jnp.sum(jnp.where(hot1, y1, 0.0), keepdims=True),
                    jnp.sum(jnp.where(hot1, x2, 0.0), keepdims=True),
                    jnp.sum(jnp.where(hot1, y2, 0.0), keepdims=True),
                    jnp.sum(jnp.where(hot2, x1, 0.0), keepdims=True),
                    jnp.sum(jnp.where(hot2, y1, 0.0), keepdims=True),
                    jnp.sum(jnp.where(hot2, x2, 0.0), keepdims=True),
                    jnp.sum(jnp.where(hot2, y2, 0.0), keepdims=True))

        def tied(_):
            pick = jnp.where(hot1, lin, jnp.int32(_N))
            oh = lin == jnp.min(pick, keepdims=True)
            b1 = (jnp.sum(jnp.where(oh, x1, 0.0), keepdims=True),
                  jnp.sum(jnp.where(oh, y1, 0.0), keepdims=True),
                  jnp.sum(jnp.where(oh, x2, 0.0), keepdims=True),
                  jnp.sum(jnp.where(oh, y2, 0.0), keepdims=True))
            return b1 + b1

        (b1x1, b1y1, b1x2, b1y2,
         b2x1, b2y1, b2x2, b2y2) = jax.lax.cond(tie, tied, common, 0)

        barea1 = (b1x2 - b1x1 + 1.0) * (b1y2 - b1y1 + 1.0)
        barea2 = (b2x2 - b2x1 + 1.0) * (b2y2 - b2y1 + 1.0)
        xq1 = jnp.maximum(b1x1, b2x1)
        yq1 = jnp.maximum(b1y1, b2y1)
        xq2 = jnp.minimum(b1x2, b2x2)
        yq2 = jnp.minimum(b1y2, b2y2)
        iwq = jnp.maximum(0.0, xq2 - xq1 + 1.0)
        ihq = jnp.maximum(0.0, yq2 - yq1 + 1.0)
        interq = iwq * ihq
        iouq = interq / (barea1 + barea2 - interq)
        acc2 = has2 & (iouq <= _NMS_THRESH) & jnp.logical_not(tie)

        xx1 = jnp.maximum(b1x1, x1)
        yy1 = jnp.maximum(b1y1, y1)
        xx2 = jnp.minimum(b1x2, x2)
        yy2 = jnp.minimum(b1y2, y2)
        iw = jnp.maximum(0.0, xx2 - xx1 + 1.0)
        ih = jnp.maximum(0.0, yy2 - yy1 + 1.0)
        inter = iw * ih
        iou1 = inter / (barea1 + areas - inter)
        ux1 = jnp.maximum(b2x1, x1)
        uy1 = jnp.maximum(b2y1, y1)
        ux2 = jnp.minimum(b2x2, x2)
        uy2 = jnp.minimum(b2y2, y2)
        uw = jnp.maximum(0.0, ux2 - ux1 + 1.0)
        uh = jnp.maximum(0.0, uy2 - uy1 + 1.0)
        uinter = uw * uh
        iou2 = uinter / (barea2 + areas - uinter)
        keepm = ((iou1 <= _NMS_THRESH)
                 & (jnp.logical_not(acc2) | (iou2 <= _NMS_THRESH))
                 & has1)
        new_ms = jnp.where(keepm, masked, neg_inf)
        ms_ref[...] = new_ms
        mx8n = jnp.max(new_ms.reshape(36, 8, _C), axis=0)

        cm1 = (lane512 == outcol) & has1
        cm2 = (lane512 == outcol + 1) & acc2
        zero = jnp.float32(0.0)
        ax1 = ax1 + jnp.where(cm1, b1x1, zero) + jnp.where(cm2, b2x1, zero)
        ay1 = ay1 + jnp.where(cm1, b1y1, zero) + jnp.where(cm2, b2y1, zero)
        ax2 = ax2 + jnp.where(cm1, b1x2, zero) + jnp.where(cm2, b2x2, zero)
        ay2 = ay2 + jnp.where(cm1, b1y2, zero) + jnp.where(cm2, b2y2, zero)
        n1 = jnp.where(has1, jnp.int32(1), jnp.int32(0))[0, 0]
        n2 = jnp.where(acc2, jnp.int32(1), jnp.int32(0))[0, 0]
        outcol = outcol + n1 + n2
        alive = (n1 == 1) & (outcol < _POST_NMS_TOP_N)
        return ax1, ay1, ax2, ay2, mx8n, outcol, alive

    def nms_body(i, carry):
        stA, stB = carry
        stA = jax.lax.cond(stA[6], lambda s: round2(ms0, x1A, y1A, x2A, y2A,
                                                    arA, s),
                           lambda s: s, stA)
        stB = jax.lax.cond(stB[6], lambda s: round2(ms1, x1B, y1B, x2B, y2B,
                                                    arB, s),
                           lambda s: s, stB)
        return stA, stB

    z = jnp.zeros((1, 512), jnp.float32)
    mx8A = jnp.max(ms0[...].reshape(36, 8, _C), axis=0)
    mx8B = jnp.max(ms1[...].reshape(36, 8, _C), axis=0)
    c0 = jnp.int32(0)
    t0 = jnp.bool_(True)
    (accA), (accB) = jax.lax.fori_loop(
        0, _POST_NMS_TOP_N, nms_body,
        ((z, z, z, z, mx8A, c0, t0), (z, z, z, z, mx8B, c0, t0)))

    out_ref[0] = jnp.concatenate(
        [accA[0], accA[1], accA[2], accA[3], z, z, z, z], axis=0)
    out_ref[1] = jnp.concatenate(
        [accB[0], accB[1], accB[2], accB[3], z, z, z, z], axis=0)


def kernel(probs, bbox_deltas, img_info):
    scores = probs[:, :, 1].reshape(2, _R, _C)
    dx = bbox_deltas[:, :, 0].reshape(2, _R, _C)
    dy = bbox_deltas[:, :, 1].reshape(2, _R, _C)
    dwv = bbox_deltas[:, :, 2].reshape(2, _R, _C)
    dhv = bbox_deltas[:, :, 3].reshape(2, _R, _C)
    hw = jnp.pad(img_info[0:1, 0:2] - 1.0, ((0, 7), (0, 126)))

    full_spec = pl.BlockSpec((2, _R, _C), lambda: (0, 0, 0))
    const_spec = pl.BlockSpec((_R, _C), lambda: (0, 0))
    out = pl.pallas_call(
        _rpn_body,
        in_specs=[
            pl.BlockSpec((8, 128), lambda: (0, 0)),
            full_spec, full_spec, full_spec, full_spec, full_spec,
            const_spec, const_spec, const_spec, const_spec,
        ],
        out_specs=pl.BlockSpec((2, 8, 512), lambda: (0, 0, 0)),
        out_shape=jax.ShapeDtypeStruct((2, 8, 512), jnp.float32),
        scratch_shapes=[pltpu.VMEM((_R, _C), jnp.float32)] * 2,
    )(hw, scores, dx, dy, dwv, dhv,
      jnp.asarray(_WA), jnp.asarray(_HA), jnp.asarray(_CXA), jnp.asarray(_CYA))

    boxes = jnp.transpose(out[:, 0:4, 0:_POST_NMS_TOP_N], (0, 2, 1))
    bcol = jnp.broadcast_to(
        jnp.arange(2, dtype=jnp.float32)[:, None, None], (2, _POST_NMS_TOP_N, 1))
    return jnp.concatenate([bcol, boxes], axis=2)


# speedup vs baseline: 1.3027x; 1.3027x over previous
import numpy as np
import jax
import jax.numpy as jnp
from jax.experimental import pallas as pl
from jax.experimental.pallas import tpu as pltpu

_FEATURE_STRIDES = [8]
_FEATURE_SHAPES = [(64, 64)]
_ANCHOR_SCALES = np.array([8.0, 16.0, 32.0], dtype=np.float32)
_ANCHOR_RATIOS = np.array([0.5, 1.0, 2.0], dtype=np.float32)
_PRE_NMS_TOP_N = 6000
_POST_NMS_TOP_N = 300
_NMS_THRESH = 0.7
_MIN_SIZE = 16.0
_LOG_MAX = float(np.log(1000.0 / 16.0))

_N = 36864
_R = 288
_C = 128


def _make_base_anchors(base_size, ratios, scales):
    w = h = float(base_size)
    x_ctr = y_ctr = (base_size - 1) * 0.5
    size = w * h
    size_ratios = size / ratios
    ws = np.round(np.sqrt(size_ratios))
    hs = np.round(ws * ratios)
    anchors = []
    for wi, hi in zip(ws, hs):
        for s in scales:
            wss = wi * s
            hss = hi * s
            anchors.append([x_ctr - 0.5 * (wss - 1), y_ctr - 0.5 * (hss - 1),
                            x_ctr + 0.5 * (wss - 1), y_ctr + 0.5 * (hss - 1)])
    return np.array(anchors, dtype=np.float32)


def _make_all_anchors():
    out = []
    for (fh, fw), stride in zip(_FEATURE_SHAPES, _FEATURE_STRIDES):
        base = _make_base_anchors(stride, _ANCHOR_RATIOS, _ANCHOR_SCALES)
        sx = np.arange(fw) * stride
        sy = np.arange(fh) * stride
        gx, gy = np.meshgrid(sx, sy)
        shifts = np.stack([gx.ravel(), gy.ravel(), gx.ravel(), gy.ravel()],
                          axis=1).astype(np.float32)
        out.append((shifts[:, None, :] + base[None, :, :]).reshape(-1, 4))
    return np.concatenate(out, axis=0)


_A = _make_all_anchors()
_WA = (_A[:, 2] - _A[:, 0] + np.float32(1.0)).reshape(_R, _C)
_HA = (_A[:, 3] - _A[:, 1] + np.float32(1.0)).reshape(_R, _C)
_CXA = (_A[:, 0] + np.float32(0.5) * _WA.reshape(-1)).reshape(_R, _C)
_CYA = (_A[:, 1] + np.float32(0.5) * _HA.reshape(-1)).reshape(_R, _C)


def _rpn_body(hw_ref, s_ref, dx_ref, dy_ref, dw_ref, dh_ref,
              wa_ref, ha_ref, cxa_ref, cya_ref, out_ref, ms0, ms1):
    h1 = hw_ref[0, 0]
    w1 = hw_ref[0, 1]
    wa = wa_ref[...]
    ha = ha_ref[...]

    lin = (jax.lax.broadcasted_iota(jnp.int32, (_R, _C), 0) * _C
           + jax.lax.broadcasted_iota(jnp.int32, (_R, _C), 1))

    def decode(b):
        s = s_ref[b]
        dx = dx_ref[b]
        dy = dy_ref[b]
        dw = jnp.minimum(dw_ref[b], _LOG_MAX)
        dh = jnp.minimum(dh_ref[b], _LOG_MAX)
        pcx = dx * wa + cxa_ref[...]
        pcy = dy * ha + cya_ref[...]
        pw = jnp.exp(dw) * wa
        ph = jnp.exp(dh) * ha
        x1 = jnp.clip(pcx - 0.5 * pw, 0.0, w1)
        y1 = jnp.clip(pcy - 0.5 * ph, 0.0, h1)
        x2 = jnp.clip(pcx + 0.5 * pw, 0.0, w1)
        y2 = jnp.clip(pcy + 0.5 * ph, 0.0, h1)
        ws = x2 - x1 + 1.0
        hs = y2 - y1 + 1.0
        s = jnp.where((ws >= _MIN_SIZE) & (hs >= _MIN_SIZE), s, -1.0)
        areas = ws * hs
        ki = jax.lax.bitcast_convert_type(s, jnp.int32)
        key = jnp.where(ki < 0, ki ^ jnp.int32(0x7FFFFFFF), ki)
        return s, x1, y1, x2, y2, areas, key

    sA, x1A, y1A, x2A, y2A, arA, keyA = decode(0)
    sB, x1B, y1B, x2B, y2B, arB, keyB = decode(1)

    def bs1(_, st):
        loA, hiA, loB, hiB = st
        midA = (loA & hiA) + ((loA ^ hiA) >> 1)
        midB = (loB & hiB) + ((loB ^ hiB) >> 1)
        cntA = jnp.sum(jnp.where(keyA > midA, 1.0, 0.0))
        cntB = jnp.sum(jnp.where(keyB > midB, 1.0, 0.0))
        pA = cntA < _PRE_NMS_TOP_N
        pB = cntB < _PRE_NMS_TOP_N
        return (jnp.where(pA, loA, midA), jnp.where(pA, midA, hiA),
                jnp.where(pB, loB, midB), jnp.where(pB, midB, hiB))

    ilo = jnp.int32(-2**31)
    ihi = jnp.int32(2**31 - 1)
    _, tauA, _, tauB = jax.lax.fori_loop(0, 32, bs1, (ilo, ihi, ilo, ihi))
    remA = jnp.float32(_PRE_NMS_TOP_N) - jnp.sum(jnp.where(keyA > tauA, 1.0, 0.0))
    remB = jnp.float32(_PRE_NMS_TOP_N) - jnp.sum(jnp.where(keyB > tauB, 1.0, 0.0))
    eqA = keyA == tauA
    eqB = keyB == tauB

    def bs2(_, st):
        loA, hiA, loB, hiB = st
        midA = (loA & hiA) + ((loA ^ hiA) >> 1)
        midB = (loB & hiB) + ((loB ^ hiB) >> 1)
        cntA = jnp.sum(jnp.where(eqA & (lin <= midA), 1.0, 0.0))
        cntB = jnp.sum(jnp.where(eqB & (lin <= midB), 1.0, 0.0))
        pA = cntA >= remA
        pB = cntB >= remB
        return (jnp.where(pA, loA, midA), jnp.where(pA, midA, hiA),
                jnp.where(pB, loB, midB), jnp.where(pB, midB, hiB))

    m1 = jnp.int32(-1)
    nn = jnp.int32(_N - 1)
    _, isA, _, isB = jax.lax.fori_loop(0, 17, bs2, (m1, nn, m1, nn))

    neg_inf = jnp.float32(-jnp.inf)
    ms0[...] = jnp.where((keyA > tauA) | (eqA & (lin <= isA)), sA, neg_inf)
    ms1[...] = jnp.where((keyB > tauB) | (eqB & (lin <= isB)), sB, neg_inf)

    lane512 = jax.lax.broadcasted_iota(jnp.int32, (1, 512), 1)

    def pick_phase(ms_ref, x1, y1, x2, y2, mx8):
        m = jnp.max(mx8, keepdims=True)
        has = m > neg_inf
        masked = ms_ref[...]
        hot = (masked == m) & (masked > neg_inf)
        cnt = jnp.sum(jnp.where(hot, 1.0, 0.0))
        bx1 = jnp.sum(jnp.where(hot, x1, 0.0), keepdims=True)
        by1 = jnp.sum(jnp.where(hot, y1, 0.0), keepdims=True)
        bx2 = jnp.sum(jnp.where(hot, x2, 0.0), keepdims=True)
        by2 = jnp.sum(jnp.where(hot, y2, 0.0), keepdims=True)
        return masked, has, hot, cnt, (bx1, by1, bx2, by2)

    def resolve(hot, x1, y1, x2, y2):
        pick = jnp.where(hot, lin, jnp.int32(_N))
        oh = lin == jnp.min(pick, keepdims=True)
        return (jnp.sum(jnp.where(oh, x1, 0.0), keepdims=True),
                jnp.sum(jnp.where(oh, y1, 0.0), keepdims=True),
                jnp.sum(jnp.where(oh, x2, 0.0), keepdims=True),
                jnp.sum(jnp.where(oh, y2, 0.0), keepdims=True))

    def sweep_phase(ms_ref, masked, has, cds, x1, y1, x2, y2, areas,
                    acc, i):
        ax1, ay1, ax2, ay2 = acc
        bx1, by1, bx2, by2 = cds
        barea = (bx2 - bx1 + 1.0) * (by2 - by1 + 1.0)
        xx1 = jnp.maximum(bx1, x1)
        yy1 = jnp.maximum(by1, y1)
        xx2 = jnp.minimum(bx2, x2)
        yy2 = jnp.minimum(by2, y2)
        iw = jnp.maximum(0.0, xx2 - xx1 + 1.0)
        ih = jnp.maximum(0.0, yy2 - yy1 + 1.0)
        inter = iw * ih
        iou = inter / (barea + areas - inter)
        new_ms = jnp.where((iou <= _NMS_THRESH) & has, masked, neg_inf)
        ms_ref[...] = new_ms
        mx8n = jnp.max(new_ms.reshape(36, 8, _C), axis=0)
        cm = lane512 == i
        zero = jnp.float32(0.0)
        ax1 = ax1 + jnp.where(cm & has, bx1, zero)
        ay1 = ay1 + jnp.where(cm & has, by1, zero)
        ax2 = ax2 + jnp.where(cm & has, bx2, zero)
        ay2 = ay2 + jnp.where(cm & has, by2, zero)
        return ax1, ay1, ax2, ay2, mx8n

    def nms_body(i, carry):
        (axA, ayA, bxA, byA, mx8A_), (axB, ayB, bxB, byB, mx8B_) = carry
        mskA, hasA, hotA, cntA, cdsA = pick_phase(ms0, x1A, y1A, x2A, y2A,
                                                  mx8A_)
        mskB, hasB, hotB, cntB, cdsB = pick_phase(ms1, x1B, y1B, x2B, y2B,
                                                  mx8B_)

        def fix(_):
            return (resolve(hotA, x1A, y1A, x2A, y2A)
                    + resolve(hotB, x1B, y1B, x2B, y2B))

        def keep(_):
            return cdsA + cdsB

        r = jax.lax.cond((cntA > 1.5) | (cntB > 1.5), fix, keep, 0)
        accA = sweep_phase(ms0, mskA, hasA, r[0:4], x1A, y1A, x2A, y2A,
                           arA, (axA, ayA, bxA, byA), i)
        accB = sweep_phase(ms1, mskB, hasB, r[4:8], x1B, y1B, x2B, y2B,
                           arB, (axB, ayB, bxB, byB), i)
        return accA, accB

    z = jnp.zeros((1, 512), jnp.float32)
    mx8A = jnp.max(ms0[...].reshape(36, 8, _C), axis=0)
    mx8B = jnp.max(ms1[...].reshape(36, 8, _C), axis=0)
    accA, accB = jax.lax.fori_loop(
        0, _POST_NMS_TOP_N, nms_body,
        ((z, z, z, z, mx8A), (z, z, z, z, mx8B)), unroll=4)

    out_ref[0] = jnp.concatenate(
        [accA[0], accA[1], accA[2], accA[3], z, z, z, z], axis=0)
    out_ref[1] = jnp.concatenate(
        [accB[0], accB[1], accB[2], accB[3], z, z, z, z], axis=0)


def kernel(probs, bbox_deltas, img_info):
    scores = probs[:, :, 1].reshape(2, _R, _C)
    dx = bbox_deltas[:, :, 0].reshape(2, _R, _C)
    dy = bbox_deltas[:, :, 1].reshape(2, _R, _C)
    dwv = bbox_deltas[:, :, 2].reshape(2, _R, _C)
    dhv = bbox_deltas[:, :, 3].reshape(2, _R, _C)
    hw = jnp.pad(img_info[0:1, 0:2] - 1.0, ((0, 7), (0, 126)))

    full_spec = pl.BlockSpec((2, _R, _C), lambda: (0, 0, 0))
    const_spec = pl.BlockSpec((_R, _C), lambda: (0, 0))
    out = pl.pallas_call(
        _rpn_body,
        in_specs=[
            pl.BlockSpec((8, 128), lambda: (0, 0)),
            full_spec, full_spec, full_spec, full_spec, full_spec,
            const_spec, const_spec, const_spec, const_spec,
        ],
        out_specs=pl.BlockSpec((2, 8, 512), lambda: (0, 0, 0)),
        out_shape=jax.ShapeDtypeStruct((2, 8, 512), jnp.float32),
        scratch_shapes=[pltpu.VMEM((_R, _C), jnp.float32)] * 2,
    )(hw, scores, dx, dy, dwv, dhv,
      jnp.asarray(_WA), jnp.asarray(_HA), jnp.asarray(_CXA), jnp.asarray(_CYA))

    boxes = jnp.transpose(out[:, 0:4, 0:_POST_NMS_TOP_N], (0, 2, 1))
    bcol = jnp.broadcast_to(
        jnp.arange(2, dtype=jnp.float32)[:, None, None], (2, _POST_NMS_TOP_N, 1))
    return jnp.concatenate([bcol, boxes], axis=2)


# speedup vs baseline: 1.3040x; 1.0010x over previous
import numpy as np
import jax
import jax.numpy as jnp
from jax.experimental import pallas as pl
from jax.experimental.pallas import tpu as pltpu

_FEATURE_STRIDES = [8]
_FEATURE_SHAPES = [(64, 64)]
_ANCHOR_SCALES = np.array([8.0, 16.0, 32.0], dtype=np.float32)
_ANCHOR_RATIOS = np.array([0.5, 1.0, 2.0], dtype=np.float32)
_PRE_NMS_TOP_N = 6000
_POST_NMS_TOP_N = 300
_NMS_THRESH = 0.7
_MIN_SIZE = 16.0
_LOG_MAX = float(np.log(1000.0 / 16.0))

_N = 36864
_R = 288
_C = 128


def _make_base_anchors(base_size, ratios, scales):
    w = h = float(base_size)
    x_ctr = y_ctr = (base_size - 1) * 0.5
    size = w * h
    size_ratios = size / ratios
    ws = np.round(np.sqrt(size_ratios))
    hs = np.round(ws * ratios)
    anchors = []
    for wi, hi in zip(ws, hs):
        for s in scales:
            wss = wi * s
            hss = hi * s
            anchors.append([x_ctr - 0.5 * (wss - 1), y_ctr - 0.5 * (hss - 1),
                            x_ctr + 0.5 * (wss - 1), y_ctr + 0.5 * (hss - 1)])
    return np.array(anchors, dtype=np.float32)


def _make_all_anchors():
    out = []
    for (fh, fw), stride in zip(_FEATURE_SHAPES, _FEATURE_STRIDES):
        base = _make_base_anchors(stride, _ANCHOR_RATIOS, _ANCHOR_SCALES)
        sx = np.arange(fw) * stride
        sy = np.arange(fh) * stride
        gx, gy = np.meshgrid(sx, sy)
        shifts = np.stack([gx.ravel(), gy.ravel(), gx.ravel(), gy.ravel()],
                          axis=1).astype(np.float32)
        out.append((shifts[:, None, :] + base[None, :, :]).reshape(-1, 4))
    return np.concatenate(out, axis=0)


_A = _make_all_anchors()
_WA = (_A[:, 2] - _A[:, 0] + np.float32(1.0)).reshape(_R, _C)
_HA = (_A[:, 3] - _A[:, 1] + np.float32(1.0)).reshape(_R, _C)
_CXA = (_A[:, 0] + np.float32(0.5) * _WA.reshape(-1)).reshape(_R, _C)
_CYA = (_A[:, 1] + np.float32(0.5) * _HA.reshape(-1)).reshape(_R, _C)


def _rpn_body(hw_ref, s_ref, dx_ref, dy_ref, dw_ref, dh_ref,
              wa_ref, ha_ref, cxa_ref, cya_ref, out_ref, ms0, ms1):
    h1 = hw_ref[0, 0]
    w1 = hw_ref[0, 1]
    wa = wa_ref[...]
    ha = ha_ref[...]

    lin = (jax.lax.broadcasted_iota(jnp.int32, (_R, _C), 0) * _C
           + jax.lax.broadcasted_iota(jnp.int32, (_R, _C), 1))

    def decode(b):
        s = s_ref[b]
        dx = dx_ref[b]
        dy = dy_ref[b]
        dw = jnp.minimum(dw_ref[b], _LOG_MAX)
        dh = jnp.minimum(dh_ref[b], _LOG_MAX)
        pcx = dx * wa + cxa_ref[...]
        pcy = dy * ha + cya_ref[...]
        pw = jnp.exp(dw) * wa
        ph = jnp.exp(dh) * ha
        x1 = jnp.clip(pcx - 0.5 * pw, 0.0, w1)
        y1 = jnp.clip(pcy - 0.5 * ph, 0.0, h1)
        x2 = jnp.clip(pcx + 0.5 * pw, 0.0, w1)
        y2 = jnp.clip(pcy + 0.5 * ph, 0.0, h1)
        ws = x2 - x1 + 1.0
        hs = y2 - y1 + 1.0
        s = jnp.where((ws >= _MIN_SIZE) & (hs >= _MIN_SIZE), s, -1.0)
        areas = ws * hs
        ki = jax.lax.bitcast_convert_type(s, jnp.int32)
        key = jnp.where(ki < 0, ki ^ jnp.int32(0x7FFFFFFF), ki)
        return s, x1, y1, x2, y2, areas, key

    sA, x1A, y1A, x2A, y2A, arA, keyA = decode(0)
    sB, x1B, y1B, x2B, y2B, arB, keyB = decode(1)

    def bs1(_, st):
        loA, hiA, loB, hiB = st
        midA = (loA & hiA) + ((loA ^ hiA) >> 1)
        midB = (loB & hiB) + ((loB ^ hiB) >> 1)
        cntA = jnp.sum(jnp.where(keyA > midA, 1.0, 0.0))
        cntB = jnp.sum(jnp.where(keyB > midB, 1.0, 0.0))
        pA = cntA < _PRE_NMS_TOP_N
        pB = cntB < _PRE_NMS_TOP_N
        return (jnp.where(pA, loA, midA), jnp.where(pA, midA, hiA),
                jnp.where(pB, loB, midB), jnp.where(pB, midB, hiB))

    ilo = jnp.int32(-2**31)
    ihi = jnp.int32(2**31 - 1)
    _, tauA, _, tauB = jax.lax.fori_loop(0, 32, bs1, (ilo, ihi, ilo, ihi))
    remA = jnp.float32(_PRE_NMS_TOP_N) - jnp.sum(jnp.where(keyA > tauA, 1.0, 0.0))
    remB = jnp.float32(_PRE_NMS_TOP_N) - jnp.sum(jnp.where(keyB > tauB, 1.0, 0.0))
    eqA = keyA == tauA
    eqB = keyB == tauB

    def bs2(_, st):
        loA, hiA, loB, hiB = st
        midA = (loA & hiA) + ((loA ^ hiA) >> 1)
        midB = (loB & hiB) + ((loB ^ hiB) >> 1)
        cntA = jnp.sum(jnp.where(eqA & (lin <= midA), 1.0, 0.0))
        cntB = jnp.sum(jnp.where(eqB & (lin <= midB), 1.0, 0.0))
        pA = cntA >= remA
        pB = cntB >= remB
        return (jnp.where(pA, loA, midA), jnp.where(pA, midA, hiA),
                jnp.where(pB, loB, midB), jnp.where(pB, midB, hiB))

    m1 = jnp.int32(-1)
    nn = jnp.int32(_N - 1)
    _, isA, _, isB = jax.lax.fori_loop(0, 17, bs2, (m1, nn, m1, nn))

    neg_inf = jnp.float32(-jnp.inf)
    ms0[...] = jnp.where((keyA > tauA) | (eqA & (lin <= isA)), sA, neg_inf)
    ms1[...] = jnp.where((keyB > tauB) | (eqB & (lin <= isB)), sB, neg_inf)

    lane512 = jax.lax.broadcasted_iota(jnp.int32, (1, 512), 1)

    def pick_phase(ms_ref, x1, y1, x2, y2, mx8):
        m = jnp.max(mx8, keepdims=True)
        has = m > neg_inf
        masked = ms_ref[...]
        hot = (masked == m) & (masked > neg_inf)
        cnt = jnp.sum(jnp.where(hot, 1.0, 0.0))
        bx1 = jnp.sum(jnp.where(hot, x1, 0.0), keepdims=True)
        by1 = jnp.sum(jnp.where(hot, y1, 0.0), keepdims=True)
        bx2 = jnp.sum(jnp.where(hot, x2, 0.0), keepdims=True)
        by2 = jnp.sum(jnp.where(hot, y2, 0.0), keepdims=True)
        return masked, has, hot, cnt, (bx1, by1, bx2, by2)

    def resolve(hot, x1, y1, x2, y2):
        pick = jnp.where(hot, lin, jnp.int32(_N))
        oh = lin == jnp.min(pick, keepdims=True)
        return (jnp.sum(jnp.where(oh, x1, 0.0), keepdims=True),
                jnp.sum(jnp.where(oh, y1, 0.0), keepdims=True),
                jnp.sum(jnp.where(oh, x2, 0.0), keepdims=True),
                jnp.sum(jnp.where(oh, y2, 0.0), keepdims=True))

    def sweep_phase(ms_ref, masked, has, cds, x1, y1, x2, y2, areas,
                    acc, i):
        ax1, ay1, ax2, ay2 = acc
        bx1, by1, bx2, by2 = cds
        barea = (bx2 - bx1 + 1.0) * (by2 - by1 + 1.0)
        xx1 = jnp.maximum(bx1, x1)
        yy1 = jnp.maximum(by1, y1)
        xx2 = jnp.minimum(bx2, x2)
        yy2 = jnp.minimum(by2, y2)
        iw = jnp.maximum(0.0, xx2 - xx1 + 1.0)
        ih = jnp.maximum(0.0, yy2 - yy1 + 1.0)
        inter = iw * ih
        iou = inter / (barea + areas - inter)
        new_ms = jnp.where((iou <= _NMS_THRESH) & has, masked, neg_inf)
        ms_ref[...] = new_ms
        mx8n = jnp.max(new_ms.reshape(36, 8, _C), axis=0)
        cm = lane512 == i
        zero = jnp.float32(0.0)
        ax1 = ax1 + jnp.where(cm & has, bx1, zero)
        ay1 = ay1 + jnp.where(cm & has, by1, zero)
        ax2 = ax2 + jnp.where(cm & has, bx2, zero)
        ay2 = ay2 + jnp.where(cm & has, by2, zero)
        return ax1, ay1, ax2, ay2, mx8n

    def nms_body(i, carry):
        (axA, ayA, bxA, byA, mx8A_), (axB, ayB, bxB, byB, mx8B_) = carry
        mskA, hasA, hotA, cntA, cdsA = pick_phase(ms0, x1A, y1A, x2A, y2A,
                                                  mx8A_)
        mskB, hasB, hotB, cntB, cdsB = pick_phase(ms1, x1B, y1B, x2B, y2B,
                                                  mx8B_)

        def fix(_):
            return (resolve(hotA, x1A, y1A, x2A, y2A)
                    + resolve(hotB, x1B, y1B, x2B, y2B))

        def keep(_):
            return cdsA + cdsB

        r = jax.lax.cond((cntA > 1.5) | (cntB > 1.5), fix, keep, 0)
        accA = sweep_phase(ms0, mskA, hasA, r[0:4], x1A, y1A, x2A, y2A,
                           arA, (axA, ayA, bxA, byA), i)
        accB = sweep_phase(ms1, mskB, hasB, r[4:8], x1B, y1B, x2B, y2B,
                           arB, (axB, ayB, bxB, byB), i)
        return accA, accB

    z = jnp.zeros((1, 512), jnp.float32)
    mx8A = jnp.max(ms0[...].reshape(36, 8, _C), axis=0)
    mx8B = jnp.max(ms1[...].reshape(36, 8, _C), axis=0)
    accA, accB = jax.lax.fori_loop(
        0, _POST_NMS_TOP_N, nms_body,
        ((z, z, z, z, mx8A), (z, z, z, z, mx8B)), unroll=2)

    out_ref[0] = jnp.concatenate(
        [accA[0], accA[1], accA[2], accA[3], z, z, z, z], axis=0)
    out_ref[1] = jnp.concatenate(
        [accB[0], accB[1], accB[2], accB[3], z, z, z, z], axis=0)


def kernel(probs, bbox_deltas, img_info):
    scores = probs[:, :, 1].reshape(2, _R, _C)
    dx = bbox_deltas[:, :, 0].reshape(2, _R, _C)
    dy = bbox_deltas[:, :, 1].reshape(2, _R, _C)
    dwv = bbox_deltas[:, :, 2].reshape(2, _R, _C)
    dhv = bbox_deltas[:, :, 3].reshape(2, _R, _C)
    hw = jnp.pad(img_info[0:1, 0:2] - 1.0, ((0, 7), (0, 126)))

    full_spec = pl.BlockSpec((2, _R, _C), lambda: (0, 0, 0))
    const_spec = pl.BlockSpec((_R, _C), lambda: (0, 0))
    out = pl.pallas_call(
        _rpn_body,
        in_specs=[
            pl.BlockSpec((8, 128), lambda: (0, 0)),
            full_spec, full_spec, full_spec, full_spec, full_spec,
            const_spec, const_spec, const_spec, const_spec,
        ],
        out_specs=pl.BlockSpec((2, 8, 512), lambda: (0, 0, 0)),
        out_shape=jax.ShapeDtypeStruct((2, 8, 512), jnp.float32),
        scratch_shapes=[pltpu.VMEM((_R, _C), jnp.float32)] * 2,
    )(hw, scores, dx, dy, dwv, dhv,
      jnp.asarray(_WA), jnp.asarray(_HA), jnp.asarray(_CXA), jnp.asarray(_CYA))

    boxes = jnp.transpose(out[:, 0:4, 0:_POST_NMS_TOP_N], (0, 2, 1))
    bcol = jnp.broadcast_to(
        jnp.arange(2, dtype=jnp.float32)[:, None, None], (2, _POST_NMS_TOP_N, 1))
    return jnp.concatenate([bcol, boxes], axis=2)
